# baseline (device time: 30128 ns/iter reference)
import jax
import jax.numpy as jnp
from jax import lax
from jax.experimental import pallas as pl
from jax.experimental.pallas import tpu as pltpu


def kernel(x, router, W1, W2):
    t_loc, d = x.shape
    e_loc, _, f = W1.shape
    t = 2 * t_loc

    def body(x_ref, r_ref, w1_ref, w2_ref, out_ref,
             xpeer_ref, rpeer_ref, cb_ref, partial_ref,
             send_sems, recv_sems):
        my_x = lax.axis_index("x")
        my_y = lax.axis_index("y")
        my_z = lax.axis_index("z")
        peer = (1 - my_x, my_y, my_z)
        mesh_t = pl.DeviceIdType.MESH

        barrier_sem = pltpu.get_barrier_semaphore()
        pl.semaphore_signal(barrier_sem, inc=1, device_id=peer,
                            device_id_type=mesh_t)
        pl.semaphore_wait(barrier_sem, 1)

        rdma_x = pltpu.make_async_remote_copy(
            src_ref=x_ref, dst_ref=xpeer_ref,
            send_sem=send_sems.at[0], recv_sem=recv_sems.at[0],
            device_id=peer, device_id_type=mesh_t)
        rdma_x.start()
        rdma_r = pltpu.make_async_remote_copy(
            src_ref=r_ref, dst_ref=rpeer_ref,
            send_sem=send_sems.at[1], recv_sem=recv_sems.at[1],
            device_id=peer, device_id_type=mesh_t)
        rdma_r.start()
        rdma_x.wait()
        rdma_r.wait()

        X = jnp.concatenate([x_ref[...], xpeer_ref[...]], axis=0)
        gates = jnp.concatenate(
            [jnp.dot(X, r_ref[...], preferred_element_type=jnp.float32),
             jnp.dot(X, rpeer_ref[...], preferred_element_type=jnp.float32)],
            axis=1)

        eidx = lax.broadcasted_iota(jnp.int32, (t, 4), 1)
        m1 = jnp.max(gates, axis=1, keepdims=True)
        i1 = jnp.min(jnp.where(gates == m1, eidx, 4), axis=1, keepdims=True)
        masked = jnp.where(eidx == i1, -jnp.inf, gates)
        m2 = jnp.max(masked, axis=1, keepdims=True)
        i2 = jnp.min(jnp.where(masked == m2, eidx, 4), axis=1, keepdims=True)
        b = jnp.exp(m2 - m1)
        w_top = 1.0 / (1.0 + b)
        w_sec = b / (1.0 + b)

        acc = jnp.zeros((t, d), jnp.float32)
        for j in range(e_loc):
            wj = (jnp.where(i1 == j, w_top, 0.0)
                  + jnp.where(i2 == j, w_sec, 0.0))
            h = jnp.maximum(
                jnp.dot(X, w1_ref[j], preferred_element_type=jnp.float32),
                0.0)
            acc = acc + wj * jnp.dot(
                h, w2_ref[j], preferred_element_type=jnp.float32)

        cb_ref[...] = acc[t_loc:, :]
        rdma_c = pltpu.make_async_remote_copy(
            src_ref=cb_ref, dst_ref=partial_ref,
            send_sem=send_sems.at[2], recv_sem=recv_sems.at[2],
            device_id=peer, device_id_type=mesh_t)
        rdma_c.start()
        rdma_c.wait()

        out_ref[...] = acc[:t_loc, :] + partial_ref[...]

    return pl.pallas_call(
        body,
        out_shape=jax.ShapeDtypeStruct((t_loc, d), jnp.float32),
        in_specs=[pl.BlockSpec(memory_space=pltpu.VMEM)] * 4,
        out_specs=pl.BlockSpec(memory_space=pltpu.VMEM),
        scratch_shapes=[
            pltpu.VMEM((t_loc, d), jnp.float32),
            pltpu.VMEM((d, e_loc), jnp.float32),
            pltpu.VMEM((t_loc, d), jnp.float32),
            pltpu.VMEM((t_loc, d), jnp.float32),
            pltpu.SemaphoreType.DMA((3,)),
            pltpu.SemaphoreType.DMA((3,)),
        ],
        compiler_params=pltpu.CompilerParams(collective_id=0),
    )(x, router, W1, W2)
